# baseline (speedup 1.0000x reference)
import numpy as np
import jax
import jax.numpy as jnp
from jax.experimental import pallas as pl
from jax.experimental.pallas import tpu as pltpu

_DIM = 16
_K = 8
_B = 3.0
_HALF = _DIM // 2
_MINW = 1e-3
_MINH = 1e-3
_MIND = 1e-3
_LEFT = -_B + 0.5
_RIGHT = _B + 0.5
_SCALE = _RIGHT - _LEFT
_DCONST = float(np.log(np.exp(1.0 - _MIND) - 1.0))
_SENT = float(np.log(np.exp(_DCONST) - 1.0))

_PERM = np.zeros(192, dtype=np.int32)
for _d in range(_HALF):
    for _k in range(_K):
        _PERM[_k * 8 + _d] = _d * 23 + _k
        _PERM[64 + _k * 8 + _d] = _d * 23 + 8 + _k
    for _j in range(_K - 1):
        _PERM[128 + _j * 8 + _d] = _d * 23 + 16 + _j
    _PERM[128 + 7 * 8 + _d] = 184

_BN = 4096


def _softplus(z):
    return jnp.maximum(z, 0.0) + jnp.log(1.0 + jnp.exp(-jnp.abs(z)))


def _softplus_nonneg(z):
    return z + jnp.log(1.0 + jnp.exp(-z))


def _tree(op, xs):
    xs = list(xs)
    while len(xs) > 1:
        nxt = [op(xs[i], xs[i + 1]) for i in range(0, len(xs) - 1, 2)]
        if len(xs) % 2:
            nxt.append(xs[-1])
        xs = nxt
    return xs[0]


def _mlp_t(x_t, w1, b1, w2, b2, w3, b3):
    h = jnp.tanh(jnp.dot(w1, x_t, preferred_element_type=jnp.float32) + b1)
    h = jnp.tanh(jnp.dot(w2, h, preferred_element_type=jnp.float32) + b2)
    return jnp.dot(w3, h, preferred_element_type=jnp.float32) + b3


def _bin_fracs(z):
    e = [jnp.exp(t) for t in z]
    rs = (2.0 * _B) / _tree(jnp.add, e)
    e2 = [jnp.exp(t * rs) for t in e]
    rs2 = (1.0 - _MINW * _K) / _tree(jnp.add, e2)
    return [_MINW + t * rs2 for t in e2]


def _spline_t(out_t, u_t):
    zw = [out_t[k * 8:(k + 1) * 8] for k in range(_K)]
    zh = [out_t[64 + k * 8:64 + (k + 1) * 8] for k in range(_K)]
    zd = [out_t[128 + j * 8:128 + (j + 1) * 8] for j in range(_K - 1)]

    u0 = u_t
    u = jnp.clip(u0, _LEFT, _RIGHT)

    wf = _bin_fracs(zw)
    hf = _bin_fracs(zh)

    def cums(fr):
        c = [fr[0]]
        for j in range(1, _K - 1):
            c.append(c[-1] + fr[j])
        return c

    cw = cums(wf)
    ch = cums(hf)

    t = (u - _LEFT) * (1.0 / _SCALE)
    ind = [t >= c for c in cw]

    def gather(vals):
        g = vals[0]
        for k in range(1, _K):
            g = jnp.where(ind[k - 1], vals[k], g)
        return g

    icw = _SCALE * gather([0.0] + cw) + _LEFT
    ibw = _SCALE * gather(wf)
    ich = _SCALE * gather([0.0] + ch) + _LEFT
    ihh = _SCALE * gather(hf)
    ider = _MIND + _softplus_nonneg(_softplus(gather([_SENT] + zd)))
    iderp1 = _MIND + _softplus_nonneg(_softplus(gather(zd + [_SENT])))
    idel = ihh / ibw

    theta = (u - icw) / ibw
    t1mt = theta * (1.0 - theta)
    th2 = theta * theta
    num = ihh * (idel * th2 + ider * t1mt)
    den = idel + (ider + iderp1 - 2.0 * idel) * t1mt
    out_in = ich + num / den
    omt = 1.0 - theta
    dnum = idel * idel * (iderp1 * th2 + 2.0 * idel * t1mt + ider * omt * omt)
    ld_in = jnp.log(dnum) - 2.0 * jnp.log(den)

    inside = (u0 >= _LEFT) & (u0 <= _RIGHT)
    y = jnp.where(inside, out_in, u0)
    ld = jnp.sum(jnp.where(inside, ld_in, 0.0), axis=0, keepdims=True)
    return y, ld


def _nsf_block(x_ref, w11, b11, w12, b12, w13, b13,
               w21, b21, w22, b22, w23, b23, y_ref, ld_ref):
    x_t = x_ref[...].T
    low_t = x_t[0:_HALF]
    up_t = x_t[_HALF:_DIM]
    out1 = _mlp_t(low_t, w11[...], b11[...], w12[...], b12[...],
                  w13[...], b13[...])
    up_new, ld1 = _spline_t(out1, up_t)
    out2 = _mlp_t(up_new, w21[...], b21[...], w22[...], b22[...],
                  w23[...], b23[...])
    low_new, ld2 = _spline_t(out2, low_t)
    y_t = jnp.concatenate([low_new, up_new], axis=0)
    y_ref[...] = y_t.T
    ld_ref[...] = ld1 + ld2


def _prep(w3, b3):
    w3z = jnp.concatenate([w3, jnp.zeros((w3.shape[0], 1), w3.dtype)], axis=1)
    b3z = jnp.concatenate([b3, jnp.zeros((1,), b3.dtype)], axis=0)
    return w3z[:, _PERM].T, b3z[_PERM][:, None]


def _run(x, f1_w1, f1_b1, f1_w2, f1_b2, f1_w3, f1_b3,
         f2_w1, f2_b1, f2_w2, f2_b2, f2_w3, f2_b3):
    n = x.shape[0]
    w13, b13 = _prep(f1_w3, f1_b3)
    w23, b23 = _prep(f2_w3, f2_b3)
    args = (x,
            f1_w1.T, f1_b1[:, None], f1_w2.T, f1_b2[:, None], w13, b13,
            f2_w1.T, f2_b1[:, None], f2_w2.T, f2_b2[:, None], w23, b23)

    full = lambda shape: pl.BlockSpec(shape, lambda i: (0, 0))
    in_specs = [
        pl.BlockSpec((_BN, _DIM), lambda i: (i, 0)),
        full((32, _HALF)), full((32, 1)), full((32, 32)), full((32, 1)),
        full((192, 32)), full((192, 1)),
        full((32, _HALF)), full((32, 1)), full((32, 32)), full((32, 1)),
        full((192, 32)), full((192, 1)),
    ]
    out_specs = [
        pl.BlockSpec((_BN, _DIM), lambda i: (i, 0)),
        pl.BlockSpec((1, _BN), lambda i: (0, i)),
    ]
    y, ld = pl.pallas_call(
        _nsf_block,
        grid=(n // _BN,),
        in_specs=in_specs,
        out_specs=out_specs,
        out_shape=[
            jax.ShapeDtypeStruct((n, _DIM), jnp.float32),
            jax.ShapeDtypeStruct((1, n), jnp.float32),
        ],
        compiler_params=pltpu.CompilerParams(
            dimension_semantics=("arbitrary",)),
    )(*args)
    return y, ld


def kernel(x, f1_w1, f1_b1, f1_w2, f1_b2, f1_w3, f1_b3,
           f2_w1, f2_b1, f2_w2, f2_b2, f2_w3, f2_b3):
    n = x.shape[0]
    args = (x, f1_w1, f1_b1, f1_w2, f1_b2, f1_w3, f1_b3,
            f2_w1, f2_b1, f2_w2, f2_b2, f2_w3, f2_b3)
    devs = jax.devices()
    if len(devs) >= 2 and n % (2 * _BN) == 0:
        from jax.sharding import Mesh, PartitionSpec as P
        try:
            from jax import shard_map as _shard_map
        except ImportError:
            from jax.experimental.shard_map import shard_map as _shard_map
        mesh = Mesh(np.asarray(devs[:2]), ("d",))
        rep = P()
        f = _shard_map(_run, mesh=mesh,
                       in_specs=(P("d"),) + (rep,) * 12,
                       out_specs=(P("d"), P(None, "d")),
                       check_vma=False)
        y, ld = f(*args)
    else:
        y, ld = _run(*args)
    return y, ld.reshape(n)


# speedup vs baseline: 1.9163x; 1.9163x over previous
import numpy as np
import jax
import jax.numpy as jnp
from jax.experimental import pallas as pl
from jax.experimental.pallas import tpu as pltpu

_DIM = 16
_K = 8
_B = 3.0
_HALF = _DIM // 2
_MINW = 1e-3
_MINH = 1e-3
_MIND = 1e-3
_LEFT = -_B + 0.5
_RIGHT = _B + 0.5
_SCALE = _RIGHT - _LEFT
_DCONST = float(np.log(np.exp(1.0 - _MIND) - 1.0))
_SENT = float(np.log(np.exp(_DCONST) - 1.0))

_PERM = np.zeros(192, dtype=np.int32)
for _d in range(_HALF):
    for _k in range(_K):
        _PERM[_k * 8 + _d] = _d * 23 + _k
        _PERM[64 + _k * 8 + _d] = _d * 23 + 8 + _k
    for _j in range(_K - 1):
        _PERM[128 + _j * 8 + _d] = _d * 23 + 16 + _j
    _PERM[128 + 7 * 8 + _d] = 184

_BN = 8192


def _softplus(z):
    return jnp.maximum(z, 0.0) + jnp.log(1.0 + jnp.exp(-jnp.abs(z)))


def _softplus_nonneg(z):
    return z + jnp.log(1.0 + jnp.exp(-z))


def _tree(op, xs):
    xs = list(xs)
    while len(xs) > 1:
        nxt = [op(xs[i], xs[i + 1]) for i in range(0, len(xs) - 1, 2)]
        if len(xs) % 2:
            nxt.append(xs[-1])
        xs = nxt
    return xs[0]


def _mlp_t(x_t, w1, b1, w2, b2, w3, b3):
    h = jnp.tanh(jnp.dot(w1, x_t, preferred_element_type=jnp.float32) + b1)
    h = jnp.tanh(jnp.dot(w2, h, preferred_element_type=jnp.float32) + b2)
    return jnp.dot(w3, h, preferred_element_type=jnp.float32) + b3


def _bin_fracs(z):
    e = [jnp.exp(t) for t in z]
    rs = (2.0 * _B) / _tree(jnp.add, e)
    e2 = [jnp.exp(t * rs) for t in e]
    rs2 = (1.0 - _MINW * _K) / _tree(jnp.add, e2)
    return [_MINW + t * rs2 for t in e2]


def _spline_t(out_t, u_t):
    zw = [out_t[k * 8:(k + 1) * 8] for k in range(_K)]
    zh = [out_t[64 + k * 8:64 + (k + 1) * 8] for k in range(_K)]
    zd = [out_t[128 + j * 8:128 + (j + 1) * 8] for j in range(_K - 1)]

    u0 = u_t
    u = jnp.clip(u0, _LEFT, _RIGHT)

    wf = _bin_fracs(zw)
    hf = _bin_fracs(zh)

    def cums(fr):
        c = [fr[0]]
        for j in range(1, _K - 1):
            c.append(c[-1] + fr[j])
        return c

    cw = cums(wf)
    ch = cums(hf)

    t = (u - _LEFT) * (1.0 / _SCALE)
    ind = [t >= c for c in cw]

    def gather(vals):
        g = vals[0]
        for k in range(1, _K):
            g = jnp.where(ind[k - 1], vals[k], g)
        return g

    icw = _SCALE * gather([0.0] + cw) + _LEFT
    ibw = _SCALE * gather(wf)
    ich = _SCALE * gather([0.0] + ch) + _LEFT
    ihh = _SCALE * gather(hf)
    ider = _MIND + _softplus_nonneg(_softplus(gather([_SENT] + zd)))
    iderp1 = _MIND + _softplus_nonneg(_softplus(gather(zd + [_SENT])))
    idel = ihh / ibw

    theta = (u - icw) / ibw
    t1mt = theta * (1.0 - theta)
    th2 = theta * theta
    num = ihh * (idel * th2 + ider * t1mt)
    den = idel + (ider + iderp1 - 2.0 * idel) * t1mt
    out_in = ich + num / den
    omt = 1.0 - theta
    dnum = idel * idel * (iderp1 * th2 + 2.0 * idel * t1mt + ider * omt * omt)
    ld_in = jnp.log(dnum) - 2.0 * jnp.log(den)

    inside = (u0 >= _LEFT) & (u0 <= _RIGHT)
    y = jnp.where(inside, out_in, u0)
    ld = jnp.sum(jnp.where(inside, ld_in, 0.0), axis=0, keepdims=True)
    return y, ld


def _nsf_block(x_ref, w11, b11, w12, b12, w13, b13,
               w21, b21, w22, b22, w23, b23, y_ref, ld_ref):
    x_t = x_ref[...].T
    low_t = x_t[0:_HALF]
    up_t = x_t[_HALF:_DIM]
    out1 = _mlp_t(low_t, w11[...], b11[...], w12[...], b12[...],
                  w13[...], b13[...])
    up_new, ld1 = _spline_t(out1, up_t)
    out2 = _mlp_t(up_new, w21[...], b21[...], w22[...], b22[...],
                  w23[...], b23[...])
    low_new, ld2 = _spline_t(out2, low_t)
    y_t = jnp.concatenate([low_new, up_new], axis=0)
    y_ref[...] = y_t.T
    ld_ref[...] = ld1 + ld2


def _prep(w3, b3):
    w3z = jnp.concatenate([w3, jnp.zeros((w3.shape[0], 1), w3.dtype)], axis=1)
    b3z = jnp.concatenate([b3, jnp.zeros((1,), b3.dtype)], axis=0)
    return w3z[:, _PERM].T, b3z[_PERM][:, None]


def _run(x, f1_w1, f1_b1, f1_w2, f1_b2, f1_w3, f1_b3,
         f2_w1, f2_b1, f2_w2, f2_b2, f2_w3, f2_b3):
    n = x.shape[0]
    w13, b13 = _prep(f1_w3, f1_b3)
    w23, b23 = _prep(f2_w3, f2_b3)
    args = (x,
            f1_w1.T, f1_b1[:, None], f1_w2.T, f1_b2[:, None], w13, b13,
            f2_w1.T, f2_b1[:, None], f2_w2.T, f2_b2[:, None], w23, b23)

    full = lambda shape: pl.BlockSpec(shape, lambda i: (0, 0))
    in_specs = [
        pl.BlockSpec((_BN, _DIM), lambda i: (i, 0)),
        full((32, _HALF)), full((32, 1)), full((32, 32)), full((32, 1)),
        full((192, 32)), full((192, 1)),
        full((32, _HALF)), full((32, 1)), full((32, 32)), full((32, 1)),
        full((192, 32)), full((192, 1)),
    ]
    out_specs = [
        pl.BlockSpec((_BN, _DIM), lambda i: (i, 0)),
        pl.BlockSpec((1, _BN), lambda i: (0, i)),
    ]
    y, ld = pl.pallas_call(
        _nsf_block,
        grid=(n // _BN,),
        in_specs=in_specs,
        out_specs=out_specs,
        out_shape=[
            jax.ShapeDtypeStruct((n, _DIM), jnp.float32),
            jax.ShapeDtypeStruct((1, n), jnp.float32),
        ],
        compiler_params=pltpu.CompilerParams(
            dimension_semantics=("arbitrary",)),
    )(*args)
    return y, ld


def kernel(x, f1_w1, f1_b1, f1_w2, f1_b2, f1_w3, f1_b3,
           f2_w1, f2_b1, f2_w2, f2_b2, f2_w3, f2_b3):
    n = x.shape[0]
    args = (x, f1_w1, f1_b1, f1_w2, f1_b2, f1_w3, f1_b3,
            f2_w1, f2_b1, f2_w2, f2_b2, f2_w3, f2_b3)
    y, ld = _run(*args)
    return y, ld.reshape(n)


# speedup vs baseline: 1.9277x; 1.0060x over previous
import numpy as np
import jax
import jax.numpy as jnp
from jax.experimental import pallas as pl
from jax.experimental.pallas import tpu as pltpu

_DIM = 16
_K = 8
_B = 3.0
_HALF = _DIM // 2
_MINW = 1e-3
_MINH = 1e-3
_MIND = 1e-3
_LEFT = -_B + 0.5
_RIGHT = _B + 0.5
_SCALE = _RIGHT - _LEFT
_DCONST = float(np.log(np.exp(1.0 - _MIND) - 1.0))
_SENT = float(np.log(np.exp(_DCONST) - 1.0))

_PERM = np.zeros(192, dtype=np.int32)
for _d in range(_HALF):
    for _k in range(_K):
        _PERM[_k * 8 + _d] = _d * 23 + _k
        _PERM[64 + _k * 8 + _d] = _d * 23 + 8 + _k
    for _j in range(_K - 1):
        _PERM[128 + _j * 8 + _d] = _d * 23 + 16 + _j
    _PERM[128 + 7 * 8 + _d] = 184

_BN = 16384


def _softplus(z):
    return jnp.maximum(z, 0.0) + jnp.log(1.0 + jnp.exp(-jnp.abs(z)))


def _softplus_nonneg(z):
    return z + jnp.log(1.0 + jnp.exp(-z))


def _tree(op, xs):
    xs = list(xs)
    while len(xs) > 1:
        nxt = [op(xs[i], xs[i + 1]) for i in range(0, len(xs) - 1, 2)]
        if len(xs) % 2:
            nxt.append(xs[-1])
        xs = nxt
    return xs[0]


def _mlp_t(x_t, w1, b1, w2, b2, w3, b3):
    h = jnp.tanh(jnp.dot(w1, x_t, preferred_element_type=jnp.float32) + b1)
    h = jnp.tanh(jnp.dot(w2, h, preferred_element_type=jnp.float32) + b2)
    return jnp.dot(w3, h, preferred_element_type=jnp.float32) + b3


def _bin_fracs(z):
    e = [jnp.exp(t) for t in z]
    rs = (2.0 * _B) / _tree(jnp.add, e)
    e2 = [jnp.exp(t * rs) for t in e]
    rs2 = (1.0 - _MINW * _K) / _tree(jnp.add, e2)
    return [_MINW + t * rs2 for t in e2]


def _spline_t(out_t, u_t):
    zw = [out_t[k * 8:(k + 1) * 8] for k in range(_K)]
    zh = [out_t[64 + k * 8:64 + (k + 1) * 8] for k in range(_K)]
    zd = [out_t[128 + j * 8:128 + (j + 1) * 8] for j in range(_K - 1)]

    u0 = u_t
    u = jnp.clip(u0, _LEFT, _RIGHT)

    wf = _bin_fracs(zw)
    hf = _bin_fracs(zh)

    def cums(fr):
        c = [fr[0]]
        for j in range(1, _K - 1):
            c.append(c[-1] + fr[j])
        return c

    cw = cums(wf)
    ch = cums(hf)

    t = (u - _LEFT) * (1.0 / _SCALE)
    ind = [t >= c for c in cw]

    def gather(vals):
        g = vals[0]
        for k in range(1, _K):
            g = jnp.where(ind[k - 1], vals[k], g)
        return g

    icw = _SCALE * gather([0.0] + cw) + _LEFT
    ibw = _SCALE * gather(wf)
    ich = _SCALE * gather([0.0] + ch) + _LEFT
    ihh = _SCALE * gather(hf)
    ider = _MIND + _softplus_nonneg(_softplus(gather([_SENT] + zd)))
    iderp1 = _MIND + _softplus_nonneg(_softplus(gather(zd + [_SENT])))
    idel = ihh / ibw

    theta = (u - icw) / ibw
    t1mt = theta * (1.0 - theta)
    th2 = theta * theta
    num = ihh * (idel * th2 + ider * t1mt)
    den = idel + (ider + iderp1 - 2.0 * idel) * t1mt
    out_in = ich + num / den
    omt = 1.0 - theta
    dnum = idel * idel * (iderp1 * th2 + 2.0 * idel * t1mt + ider * omt * omt)
    ld_in = jnp.log(dnum) - 2.0 * jnp.log(den)

    inside = (u0 >= _LEFT) & (u0 <= _RIGHT)
    y = jnp.where(inside, out_in, u0)
    ld = jnp.sum(jnp.where(inside, ld_in, 0.0), axis=0, keepdims=True)
    return y, ld


def _nsf_block(x_ref, w11, b11, w12, b12, w13, b13,
               w21, b21, w22, b22, w23, b23, y_ref, ld_ref):
    x_t = x_ref[...].T
    low_t = x_t[0:_HALF]
    up_t = x_t[_HALF:_DIM]
    out1 = _mlp_t(low_t, w11[...], b11[...], w12[...], b12[...],
                  w13[...], b13[...])
    up_new, ld1 = _spline_t(out1, up_t)
    out2 = _mlp_t(up_new, w21[...], b21[...], w22[...], b22[...],
                  w23[...], b23[...])
    low_new, ld2 = _spline_t(out2, low_t)
    y_t = jnp.concatenate([low_new, up_new], axis=0)
    y_ref[...] = y_t.T
    ld_ref[...] = ld1 + ld2


def _prep(w3, b3):
    w3z = jnp.concatenate([w3, jnp.zeros((w3.shape[0], 1), w3.dtype)], axis=1)
    b3z = jnp.concatenate([b3, jnp.zeros((1,), b3.dtype)], axis=0)
    return w3z[:, _PERM].T, b3z[_PERM][:, None]


def _run(x, f1_w1, f1_b1, f1_w2, f1_b2, f1_w3, f1_b3,
         f2_w1, f2_b1, f2_w2, f2_b2, f2_w3, f2_b3):
    n = x.shape[0]
    w13, b13 = _prep(f1_w3, f1_b3)
    w23, b23 = _prep(f2_w3, f2_b3)
    args = (x,
            f1_w1.T, f1_b1[:, None], f1_w2.T, f1_b2[:, None], w13, b13,
            f2_w1.T, f2_b1[:, None], f2_w2.T, f2_b2[:, None], w23, b23)

    full = lambda shape: pl.BlockSpec(shape, lambda i: (0, 0))
    in_specs = [
        pl.BlockSpec((_BN, _DIM), lambda i: (i, 0)),
        full((32, _HALF)), full((32, 1)), full((32, 32)), full((32, 1)),
        full((192, 32)), full((192, 1)),
        full((32, _HALF)), full((32, 1)), full((32, 32)), full((32, 1)),
        full((192, 32)), full((192, 1)),
    ]
    out_specs = [
        pl.BlockSpec((_BN, _DIM), lambda i: (i, 0)),
        pl.BlockSpec((1, _BN), lambda i: (0, i)),
    ]
    y, ld = pl.pallas_call(
        _nsf_block,
        grid=(n // _BN,),
        in_specs=in_specs,
        out_specs=out_specs,
        out_shape=[
            jax.ShapeDtypeStruct((n, _DIM), jnp.float32),
            jax.ShapeDtypeStruct((1, n), jnp.float32),
        ],
        compiler_params=pltpu.CompilerParams(
            dimension_semantics=("arbitrary",)),
    )(*args)
    return y, ld


def kernel(x, f1_w1, f1_b1, f1_w2, f1_b2, f1_w3, f1_b3,
           f2_w1, f2_b1, f2_w2, f2_b2, f2_w3, f2_b3):
    n = x.shape[0]
    args = (x, f1_w1, f1_b1, f1_w2, f1_b2, f1_w3, f1_b3,
            f2_w1, f2_b1, f2_w2, f2_b2, f2_w3, f2_b3)
    y, ld = _run(*args)
    return y, ld.reshape(n)


# speedup vs baseline: 3.2724x; 1.6975x over previous
import numpy as np
import jax
import jax.numpy as jnp
from jax.experimental import pallas as pl
from jax.experimental.pallas import tpu as pltpu

_DIM = 16
_K = 8
_B = 3.0
_HALF = _DIM // 2
_MINW = 1e-3
_MINH = 1e-3
_MIND = 1e-3
_LEFT = -_B + 0.5
_RIGHT = _B + 0.5
_SCALE = _RIGHT - _LEFT
_DCONST = float(np.log(np.exp(1.0 - _MIND) - 1.0))
_SENT = float(np.log(np.exp(_DCONST) - 1.0))

_PERM = np.zeros(192, dtype=np.int32)
for _d in range(_HALF):
    for _k in range(_K):
        _PERM[_k * 8 + _d] = _d * 23 + _k
        _PERM[64 + _k * 8 + _d] = _d * 23 + 8 + _k
    for _j in range(_K - 1):
        _PERM[128 + _j * 8 + _d] = _d * 23 + 16 + _j
    _PERM[128 + 7 * 8 + _d] = 184

_BN = 16384


def _softplus(z):
    return jnp.maximum(z, 0.0) + jnp.log(1.0 + jnp.exp(-jnp.abs(z)))


def _softplus_nonneg(z):
    return z + jnp.log(1.0 + jnp.exp(-z))


def _tree(op, xs):
    xs = list(xs)
    while len(xs) > 1:
        nxt = [op(xs[i], xs[i + 1]) for i in range(0, len(xs) - 1, 2)]
        if len(xs) % 2:
            nxt.append(xs[-1])
        xs = nxt
    return xs[0]


def _mlp_t(x_t, w1, b1, w2, b2, w3, b3):
    h = jnp.tanh(jnp.dot(w1, x_t, preferred_element_type=jnp.float32) + b1)
    h = jnp.tanh(jnp.dot(w2, h, preferred_element_type=jnp.float32) + b2)
    return jnp.dot(w3, h, preferred_element_type=jnp.float32) + b3


def _bin_fracs(z):
    e = [jnp.exp(t) for t in z]
    rs = (2.0 * _B) / _tree(jnp.add, e)
    e2 = [jnp.exp(t * rs) for t in e]
    rs2 = (1.0 - _MINW * _K) / _tree(jnp.add, e2)
    return [_MINW + t * rs2 for t in e2]


def _spline_t(out_t, u_t):
    zw = [out_t[k * 8:(k + 1) * 8] for k in range(_K)]
    zh = [out_t[64 + k * 8:64 + (k + 1) * 8] for k in range(_K)]
    zd = [out_t[128 + j * 8:128 + (j + 1) * 8] for j in range(_K - 1)]

    u0 = u_t
    u = jnp.clip(u0, _LEFT, _RIGHT)

    wf = _bin_fracs(zw)
    hf = _bin_fracs(zh)

    def cums(fr):
        c = [fr[0]]
        for j in range(1, _K - 1):
            c.append(c[-1] + fr[j])
        return c

    cw = cums(wf)
    ch = cums(hf)

    t = (u - _LEFT) * (1.0 / _SCALE)
    ind = [t >= c for c in cw]

    def gather(vals):
        g = vals[0]
        for k in range(1, _K):
            g = jnp.where(ind[k - 1], vals[k], g)
        return g

    icw = _SCALE * gather([0.0] + cw) + _LEFT
    ibw = _SCALE * gather(wf)
    ich = _SCALE * gather([0.0] + ch) + _LEFT
    ihh = _SCALE * gather(hf)
    ider = _MIND + _softplus_nonneg(_softplus(gather([_SENT] + zd)))
    iderp1 = _MIND + _softplus_nonneg(_softplus(gather(zd + [_SENT])))
    idel = ihh / ibw

    theta = (u - icw) / ibw
    t1mt = theta * (1.0 - theta)
    th2 = theta * theta
    num = ihh * (idel * th2 + ider * t1mt)
    den = idel + (ider + iderp1 - 2.0 * idel) * t1mt
    out_in = ich + num / den
    omt = 1.0 - theta
    dnum = idel * idel * (iderp1 * th2 + 2.0 * idel * t1mt + ider * omt * omt)
    ld_in = jnp.log(dnum) - 2.0 * jnp.log(den)

    inside = (u0 >= _LEFT) & (u0 <= _RIGHT)
    y = jnp.where(inside, out_in, u0)
    ld = jnp.sum(jnp.where(inside, ld_in, 0.0), axis=0, keepdims=True)
    return y, ld


def _nsf_block(x_ref, w11, b11, w12, b12, w13, b13,
               w21, b21, w22, b22, w23, b23, y_ref, ld_ref):
    x_t = x_ref[...]
    low_t = x_t[0:_HALF]
    up_t = x_t[_HALF:_DIM]
    out1 = _mlp_t(low_t, w11[...], b11[...], w12[...], b12[...],
                  w13[...], b13[...])
    up_new, ld1 = _spline_t(out1, up_t)
    out2 = _mlp_t(up_new, w21[...], b21[...], w22[...], b22[...],
                  w23[...], b23[...])
    low_new, ld2 = _spline_t(out2, low_t)
    y_ref[...] = jnp.concatenate([low_new, up_new], axis=0)
    ld_ref[...] = ld1 + ld2


def _prep(w3, b3):
    w3z = jnp.concatenate([w3, jnp.zeros((w3.shape[0], 1), w3.dtype)], axis=1)
    b3z = jnp.concatenate([b3, jnp.zeros((1,), b3.dtype)], axis=0)
    return w3z[:, _PERM].T, b3z[_PERM][:, None]


def _run(x, f1_w1, f1_b1, f1_w2, f1_b2, f1_w3, f1_b3,
         f2_w1, f2_b1, f2_w2, f2_b2, f2_w3, f2_b3):
    n = x.shape[0]
    w13, b13 = _prep(f1_w3, f1_b3)
    w23, b23 = _prep(f2_w3, f2_b3)
    args = (x.T,
            f1_w1.T, f1_b1[:, None], f1_w2.T, f1_b2[:, None], w13, b13,
            f2_w1.T, f2_b1[:, None], f2_w2.T, f2_b2[:, None], w23, b23)

    full = lambda shape: pl.BlockSpec(shape, lambda i: (0, 0))
    in_specs = [
        pl.BlockSpec((_DIM, _BN), lambda i: (0, i)),
        full((32, _HALF)), full((32, 1)), full((32, 32)), full((32, 1)),
        full((192, 32)), full((192, 1)),
        full((32, _HALF)), full((32, 1)), full((32, 32)), full((32, 1)),
        full((192, 32)), full((192, 1)),
    ]
    out_specs = [
        pl.BlockSpec((_DIM, _BN), lambda i: (0, i)),
        pl.BlockSpec((1, _BN), lambda i: (0, i)),
    ]
    yt, ld = pl.pallas_call(
        _nsf_block,
        grid=(n // _BN,),
        in_specs=in_specs,
        out_specs=out_specs,
        out_shape=[
            jax.ShapeDtypeStruct((_DIM, n), jnp.float32),
            jax.ShapeDtypeStruct((1, n), jnp.float32),
        ],
        compiler_params=pltpu.CompilerParams(
            dimension_semantics=("arbitrary",)),
    )(*args)
    return yt.T, ld


def kernel(x, f1_w1, f1_b1, f1_w2, f1_b2, f1_w3, f1_b3,
           f2_w1, f2_b1, f2_w2, f2_b2, f2_w3, f2_b3):
    n = x.shape[0]
    args = (x, f1_w1, f1_b1, f1_w2, f1_b2, f1_w3, f1_b3,
            f2_w1, f2_b1, f2_w2, f2_b2, f2_w3, f2_b3)
    y, ld = _run(*args)
    return y, ld.reshape(n)
